# baseline (speedup 1.0000x reference)
import functools

import jax
import jax.numpy as jnp
from jax import lax
from jax.experimental import pallas as pl
from jax.experimental.pallas import tpu as pltpu

B, N, C, H, W = 2, 4, 16, 512, 512
W_SPARSE, W_DENSE, W_SEM = 1.0, 0.05, 1.0

BN = B * N
HB = 64
LOG2E = 1.4426950408889634


def _loss_kernel(sp_ref, gt_ref, dp_ref, sg_ref, dg_ref, out_ref):
    gt = gt_ref[0]

    m = sp_ref[0, 0]
    for c in range(1, C):
        m = jnp.maximum(m, sp_ref[0, c])

    mscaled = m * LOG2E
    e = jnp.zeros((HB, W), jnp.float32)
    logit_gt = jnp.zeros((HB, W), jnp.float32)
    for c in range(C):
        xc = sp_ref[0, c]
        e = e + jnp.exp2(xc * LOG2E - mscaled)
        logit_gt = logit_gt + jnp.where(gt == c, xc, 0.0)
    nll = jnp.log(e) + m - logit_gt

    mf = (gt > 0).astype(jnp.float32)
    nll_sum = jnp.sum(nll * mf)
    cnt = jnp.sum(mf)

    dpv = dp_ref[0]
    sgv = sg_ref[0]
    dgv = dg_ref[0]
    ms = mf * (sgv > 0.0).astype(jnp.float32)
    md = mf * (dgv > 0.0).astype(jnp.float32)
    l1s = jnp.sum(jnp.abs(dpv - sgv) * ms)
    l1d = jnp.sum(jnp.abs(dpv - dgv) * md)
    cnt_s = jnp.sum(ms)
    cnt_d = jnp.sum(md)

    lane = lax.broadcasted_iota(jnp.int32, (1, 128), 1)
    vec = (jnp.where(lane == 0, nll_sum, 0.0)
           + jnp.where(lane == 1, cnt, 0.0)
           + jnp.where(lane == 2, l1s, 0.0)
           + jnp.where(lane == 3, cnt_s, 0.0)
           + jnp.where(lane == 4, l1d, 0.0)
           + jnp.where(lane == 5, cnt_d, 0.0))

    first = jnp.logical_and(pl.program_id(0) == 0, pl.program_id(1) == 0)

    @pl.when(first)
    def _init():
        out_ref[...] = vec

    @pl.when(jnp.logical_not(first))
    def _acc():
        out_ref[...] += vec


@functools.partial(jax.jit, static_argnames=())
def kernel(depth_pred, semantic_pred, sparse_depth_gt, dense_depth_gt, semantic_gt):
    sp = semantic_pred.reshape(BN, C, H, W)
    gt = semantic_gt.reshape(BN, H, W).astype(jnp.int32)
    dp = depth_pred.reshape(BN, H, W)
    sg = sparse_depth_gt.reshape(BN, H, W)
    dg = dense_depth_gt.reshape(BN, H, W)

    nh = H // HB
    acc = pl.pallas_call(
        _loss_kernel,
        grid=(BN, nh),
        in_specs=[
            pl.BlockSpec((1, C, HB, W), lambda b, h: (b, 0, h, 0)),
            pl.BlockSpec((1, HB, W), lambda b, h: (b, h, 0)),
            pl.BlockSpec((1, HB, W), lambda b, h: (b, h, 0)),
            pl.BlockSpec((1, HB, W), lambda b, h: (b, h, 0)),
            pl.BlockSpec((1, HB, W), lambda b, h: (b, h, 0)),
        ],
        out_specs=pl.BlockSpec((1, 128), lambda b, h: (0, 0)),
        out_shape=jax.ShapeDtypeStruct((1, 128), jnp.float32),
    )(sp, gt, dp, sg, dg)

    nll_sum, cnt, l1s, cnt_s, l1d, cnt_d = (
        acc[0, 0], acc[0, 1], acc[0, 2], acc[0, 3], acc[0, 4], acc[0, 5])

    l_d = jnp.where(cnt_s > 0, l1s / jnp.maximum(cnt_s, 1.0), 0.0)
    l_pd = jnp.where(cnt_d > 0, l1d / jnp.maximum(cnt_d, 1.0), 0.0)
    l_sem = jnp.where(cnt > 0, nll_sum / jnp.maximum(cnt, 1.0), 0.0)
    return W_SPARSE * l_d + W_DENSE * l_pd + W_SEM * l_sem


# speedup vs baseline: 1.1194x; 1.1194x over previous
import functools

import jax
import jax.numpy as jnp
from jax import lax
from jax.experimental import pallas as pl
from jax.experimental.pallas import tpu as pltpu

B, N, C, H, W = 2, 4, 16, 512, 512
W_SPARSE, W_DENSE, W_SEM = 1.0, 0.05, 1.0

BN = B * N
HB = 128
LOG2E = 1.4426950408889634


def _loss_kernel(sp_ref, gt_ref, dp_ref, sg_ref, dg_ref, out_ref):
    gt = gt_ref[0]

    m = sp_ref[0, 0]
    for c in range(1, C):
        m = jnp.maximum(m, sp_ref[0, c])

    mscaled = m * LOG2E
    e = jnp.zeros((HB, W), jnp.float32)
    logit_gt = jnp.zeros((HB, W), jnp.float32)
    for c in range(C):
        xc = sp_ref[0, c]
        e = e + jnp.exp2(xc * LOG2E - mscaled)
        logit_gt = logit_gt + jnp.where(gt == c, xc, 0.0)
    nll = jnp.log(e) + m - logit_gt

    mf = (gt > 0).astype(jnp.float32)
    nll_sum = jnp.sum(nll * mf)
    cnt = jnp.sum(mf)

    dpv = dp_ref[0]
    sgv = sg_ref[0]
    dgv = dg_ref[0]
    ms = mf * (sgv > 0.0).astype(jnp.float32)
    md = mf * (dgv > 0.0).astype(jnp.float32)
    l1s = jnp.sum(jnp.abs(dpv - sgv) * ms)
    l1d = jnp.sum(jnp.abs(dpv - dgv) * md)
    cnt_s = jnp.sum(ms)
    cnt_d = jnp.sum(md)

    lane = lax.broadcasted_iota(jnp.int32, (1, 128), 1)
    vec = (jnp.where(lane == 0, nll_sum, 0.0)
           + jnp.where(lane == 1, cnt, 0.0)
           + jnp.where(lane == 2, l1s, 0.0)
           + jnp.where(lane == 3, cnt_s, 0.0)
           + jnp.where(lane == 4, l1d, 0.0)
           + jnp.where(lane == 5, cnt_d, 0.0))

    first = jnp.logical_and(pl.program_id(0) == 0, pl.program_id(1) == 0)

    @pl.when(first)
    def _init():
        out_ref[...] = vec

    @pl.when(jnp.logical_not(first))
    def _acc():
        out_ref[...] += vec


@functools.partial(jax.jit, static_argnames=())
def kernel(depth_pred, semantic_pred, sparse_depth_gt, dense_depth_gt, semantic_gt):
    sp = semantic_pred.reshape(BN, C, H, W)
    gt = semantic_gt.reshape(BN, H, W).astype(jnp.int32)
    dp = depth_pred.reshape(BN, H, W)
    sg = sparse_depth_gt.reshape(BN, H, W)
    dg = dense_depth_gt.reshape(BN, H, W)

    nh = H // HB
    acc = pl.pallas_call(
        _loss_kernel,
        grid=(BN, nh),
        in_specs=[
            pl.BlockSpec((1, C, HB, W), lambda b, h: (b, 0, h, 0)),
            pl.BlockSpec((1, HB, W), lambda b, h: (b, h, 0)),
            pl.BlockSpec((1, HB, W), lambda b, h: (b, h, 0)),
            pl.BlockSpec((1, HB, W), lambda b, h: (b, h, 0)),
            pl.BlockSpec((1, HB, W), lambda b, h: (b, h, 0)),
        ],
        out_specs=pl.BlockSpec((1, 128), lambda b, h: (0, 0)),
        out_shape=jax.ShapeDtypeStruct((1, 128), jnp.float32),
    )(sp, gt, dp, sg, dg)

    nll_sum, cnt, l1s, cnt_s, l1d, cnt_d = (
        acc[0, 0], acc[0, 1], acc[0, 2], acc[0, 3], acc[0, 4], acc[0, 5])

    l_d = jnp.where(cnt_s > 0, l1s / jnp.maximum(cnt_s, 1.0), 0.0)
    l_pd = jnp.where(cnt_d > 0, l1d / jnp.maximum(cnt_d, 1.0), 0.0)
    l_sem = jnp.where(cnt > 0, nll_sum / jnp.maximum(cnt, 1.0), 0.0)
    return W_SPARSE * l_d + W_DENSE * l_pd + W_SEM * l_sem
